# baseline (speedup 1.0000x reference)
import functools

import jax
import jax.numpy as jnp
from jax import lax
from jax.experimental import pallas as pl
from jax.experimental.pallas import tpu as pltpu
from jax.experimental.pallas import tpu_sc as plsc

E = 8
TOP_K = 2
H = 1024
FF = 2048
ALPHA = 1.702
LIMIT = 7.0
T = 2048

BM = 128
P_PAD = TOP_K * T + E * BM
M_TILES = P_PAD // BM

NC, NS = 2, 16
NW = NC * NS
GCHUNK = 32
H_HALF = H // 2


def _router_body(x_ref, rw_ref, rb_ref, score_ref, idx_ref, p_ref):
    x = x_ref[...]
    logits = lax.dot_general(x, rw_ref[...], (((1,), (1,)), ((), ())),
                             preferred_element_type=jnp.float32) + rb_ref[...]
    iota = lax.broadcasted_iota(jnp.int32, (T, E), 1)
    m1 = jnp.max(logits, axis=1, keepdims=True)
    a1 = jnp.min(jnp.where(logits == m1, iota, E), axis=1, keepdims=True)
    l2 = jnp.where(iota == a1, -jnp.inf, logits)
    m2 = jnp.max(l2, axis=1, keepdims=True)
    a2 = jnp.min(jnp.where(l2 == m2, iota, E), axis=1, keepdims=True)
    ex = jnp.exp(m2 - m1)
    p1 = 1.0 / (1.0 + ex)
    p2 = ex / (1.0 + ex)
    score_ref[...] = jnp.where(iota == a1, p1, 0.0) + jnp.where(iota == a2, p2, 0.0)
    idx_ref[...] = jnp.concatenate([a1, a2], axis=1)
    p_ref[...] = jnp.concatenate([p1, p2], axis=1)


def _router(x, router_w, router_b):
    return pl.pallas_call(
        _router_body,
        out_shape=(
            jax.ShapeDtypeStruct((T, E), jnp.float32),
            jax.ShapeDtypeStruct((T, TOP_K), jnp.int32),
            jax.ShapeDtypeStruct((T, TOP_K), jnp.float32),
        ),
    )(x, router_w, router_b)


def _sc_row_gather(table, idx, n_rows):
    mesh = plsc.VectorSubcoreMesh(core_axis_name="c", subcore_axis_name="s",
                                  num_cores=NC, num_subcores=NS)
    rows_per_w = n_rows // NW

    nchunks = rows_per_w // GCHUNK

    @functools.partial(
        pl.kernel,
        out_type=jax.ShapeDtypeStruct((n_rows, H), jnp.float32),
        mesh=mesh,
        scratch_types=[
            [pltpu.VMEM((GCHUNK,), jnp.int32)] * 2,
            [pltpu.VMEM((GCHUNK, H), jnp.float32)] * 2,
            [pltpu.SemaphoreType.DMA] * 2,
            [pltpu.SemaphoreType.DMA] * 2,
        ],
    )
    def k(x_hbm, i_hbm, out_hbm, idx_v, rows_v, gsem, wsem):
        wid = lax.axis_index("s") * NC + lax.axis_index("c")
        base0 = wid * rows_per_w

        gd = [None, None]
        wd = [None, None]
        for i in range(nchunks):
            b = i % 2
            base = base0 + i * GCHUNK
            if i >= 2:
                wd[b].wait()
            pltpu.sync_copy(i_hbm.at[pl.ds(base, GCHUNK)], idx_v[b])
            gd[b] = pltpu.async_copy(x_hbm.at[idx_v[b]], rows_v[b], gsem[b])
            if i >= 1:
                pb = (i - 1) % 2
                gd[pb].wait()
                wd[pb] = pltpu.async_copy(
                    rows_v[pb],
                    out_hbm.at[pl.ds(base0 + (i - 1) * GCHUNK, GCHUNK)],
                    wsem[pb])
        lb = (nchunks - 1) % 2
        gd[lb].wait()
        pltpu.sync_copy(rows_v[lb],
                        out_hbm.at[pl.ds(base0 + (nchunks - 1) * GCHUNK, GCHUNK)])
        if nchunks >= 2:
            wd[(nchunks - 2) % 2].wait()

    return k(table, idx)


def _sc_dispatch(x, pos):
    mesh = plsc.VectorSubcoreMesh(core_axis_name="c", subcore_axis_name="s",
                                  num_cores=NC, num_subcores=NS)
    n_pairs = TOP_K * T
    rows_per_w = n_pairs // NW
    nchunks = rows_per_w // GCHUNK

    @functools.partial(
        pl.kernel,
        out_type=jax.ShapeDtypeStruct((P_PAD, H), jnp.float32),
        mesh=mesh,
        scratch_types=[
            [pltpu.VMEM((GCHUNK,), jnp.int32)] * 2,
            [pltpu.VMEM((GCHUNK,), jnp.int32)] * 2,
            [pltpu.VMEM((GCHUNK, H), jnp.float32)] * 2,
            [pltpu.SemaphoreType.DMA] * 2,
            [pltpu.SemaphoreType.DMA] * 2,
        ],
    )
    def k(x_hbm, pos_hbm, out_hbm, pos_v, idx_v, rows_v, gsem, wsem):
        wid = lax.axis_index("s") * NC + lax.axis_index("c")
        base0 = wid * rows_per_w
        gd = [None, None]
        wd = [None, None]
        for i in range(nchunks):
            b = i % 2
            base = base0 + i * GCHUNK
            if i >= 2:
                wd[b].wait()
            pltpu.sync_copy(pos_hbm.at[pl.ds(base, GCHUNK)], pos_v[b])
            v = lax.iota(jnp.int32, 16)
            for h in range(GCHUNK // 16):
                idx_v[b][pl.ds(h * 16, 16)] = lax.shift_right_logical(
                    v + (base + h * 16), 1)
            gd[b] = pltpu.async_copy(x_hbm.at[idx_v[b]], rows_v[b], gsem[b])
            if i >= 1:
                pb = (i - 1) % 2
                gd[pb].wait()
                wd[pb] = pltpu.async_copy(rows_v[pb], out_hbm.at[pos_v[pb]],
                                          wsem[pb])
        lb = (nchunks - 1) % 2
        gd[lb].wait()
        pltpu.sync_copy(rows_v[lb], out_hbm.at[pos_v[lb]])
        if nchunks >= 2:
            wd[(nchunks - 2) % 2].wait()

    return k(x, pos)


def _ffn_body(s_ref, x_ref, gw_hbm, gb_ref, uw_hbm, ub_ref,
              dw_hbm, db_ref, o_ref, gw_v, uw_v, dw_v, sems):
    i = pl.program_id(0)
    e = s_ref[i, 0]
    p = s_ref[i, 1]
    nx = s_ref[i, 2]
    prev = s_ref[jnp.maximum(i - 1, 0), 0]
    changed = jnp.logical_or(i == 0, e != prev)

    NSPL = 4
    CH_FF, CH_H = FF // NSPL, H // NSPL

    def copies(ee, bb):
        cs = []
        for q in range(NSPL):
            cs.append(pltpu.make_async_copy(
                gw_hbm.at[ee, pl.ds(q * CH_FF, CH_FF)],
                gw_v.at[bb, pl.ds(q * CH_FF, CH_FF)], sems.at[bb, 0, q]))
            cs.append(pltpu.make_async_copy(
                uw_hbm.at[ee, pl.ds(q * CH_FF, CH_FF)],
                uw_v.at[bb, pl.ds(q * CH_FF, CH_FF)], sems.at[bb, 1, q]))
            cs.append(pltpu.make_async_copy(
                dw_hbm.at[ee, pl.ds(q * CH_H, CH_H)],
                dw_v.at[bb, pl.ds(q * CH_H, CH_H)], sems.at[bb, 2, q]))
        return cs

    def start(ee, bb):
        for c in copies(ee, bb):
            c.start()

    def wait(ee, bb):
        for c in copies(ee, bb):
            c.wait()

    @pl.when(i == 0)
    def _():
        start(e, p)

    @pl.when(changed)
    def _():
        wait(e, p)

        @pl.when(nx != e)
        def _():
            start(nx, 1 - p)

    x = x_ref[...]
    gate = lax.dot_general(x, gw_v[p], (((1,), (1,)), ((), ())),
                           preferred_element_type=jnp.float32) + gb_ref[0]
    gate = jnp.minimum(gate, LIMIT)
    glu = gate * jax.nn.sigmoid(gate * ALPHA)
    up = lax.dot_general(x, uw_v[p], (((1,), (1,)), ((), ())),
                         preferred_element_type=jnp.float32) + ub_ref[0]
    up = jnp.clip(up, -LIMIT, LIMIT)
    gated = (up + 1.0) * glu
    out = lax.dot_general(gated, dw_v[p], (((1,), (1,)), ((), ())),
                          preferred_element_type=jnp.float32) + db_ref[0]
    o_ref[...] = out


def _ffn(sched, x_sorted, gate_w, gate_b, up_w, up_b, down_w, down_b):
    grid_spec = pltpu.PrefetchScalarGridSpec(
        num_scalar_prefetch=1,
        grid=(M_TILES,),
        in_specs=[
            pl.BlockSpec((BM, H), lambda i, s: (i, 0)),
            pl.BlockSpec(memory_space=pl.ANY),
            pl.BlockSpec((1, 1, FF), lambda i, s: (s[i, 0], 0, 0)),
            pl.BlockSpec(memory_space=pl.ANY),
            pl.BlockSpec((1, 1, FF), lambda i, s: (s[i, 0], 0, 0)),
            pl.BlockSpec(memory_space=pl.ANY),
            pl.BlockSpec((1, 1, H), lambda i, s: (s[i, 0], 0, 0)),
        ],
        out_specs=pl.BlockSpec((BM, H), lambda i, s: (i, 0)),
        scratch_shapes=[
            pltpu.VMEM((2, FF, H), jnp.float32),
            pltpu.VMEM((2, FF, H), jnp.float32),
            pltpu.VMEM((2, H, FF), jnp.float32),
            pltpu.SemaphoreType.DMA((2, 3, 4)),
        ],
    )
    return pl.pallas_call(
        _ffn_body,
        grid_spec=grid_spec,
        out_shape=jax.ShapeDtypeStruct((P_PAD, H), jnp.float32),
    )(sched, x_sorted,
      gate_w, gate_b.reshape(E, 1, FF),
      up_w, up_b.reshape(E, 1, FF),
      down_w, down_b.reshape(E, 1, H))


BN_ADD = 256


def _add_body(a_ref, b_ref, p_ref, o_ref):
    p = p_ref[...]
    o_ref[...] = a_ref[...] * p[:, 0:1] + b_ref[...] * p[:, 1:2]


def _combine_add(g, top_p):
    nblk = T // BN_ADD
    return pl.pallas_call(
        _add_body,
        grid=(nblk,),
        in_specs=[
            pl.BlockSpec((BN_ADD, H), lambda i: (i, 0)),
            pl.BlockSpec((BN_ADD, H), lambda i: (i + nblk, 0)),
            pl.BlockSpec((BN_ADD, TOP_K), lambda i: (i, 0)),
        ],
        out_specs=pl.BlockSpec((BN_ADD, H), lambda i: (i, 0)),
        out_shape=jax.ShapeDtypeStruct((T, H), jnp.float32),
    )(g, g, top_p)


def kernel(hidden_states, router_w, router_b, gate_w, gate_b, up_w, up_b,
           down_w, down_b):
    x = hidden_states.reshape(T, H)
    score, top_idx, top_p = _router(x, router_w, router_b.reshape(1, E))

    pairs = top_idx.reshape(-1)
    onehot = (pairs[:, None] == jnp.arange(E)[None, :]).astype(jnp.int32)
    counts = onehot.sum(0)
    rank = jnp.take_along_axis(jnp.cumsum(onehot, axis=0) - onehot,
                               pairs[:, None], axis=1)[:, 0]
    aligned = ((counts + BM - 1) // BM) * BM
    bounds = jnp.cumsum(aligned)
    off = bounds - aligned
    pos = off[pairs] + rank
    tile_start = jnp.arange(M_TILES, dtype=jnp.int32) * BM
    eft = jnp.minimum(
        (tile_start[:, None] >= bounds[None, :]).astype(jnp.int32).sum(1), E - 1)
    changes = jnp.concatenate([jnp.ones((1,), bool), eft[1:] != eft[:-1]])
    seg_id = jnp.cumsum(changes.astype(jnp.int32)) - 1
    par = seg_id % 2
    seg_expert = jnp.zeros((M_TILES,), jnp.int32).at[seg_id].set(eft)
    nxt = seg_expert[jnp.minimum(seg_id + 1, seg_id[-1])]
    sched = jnp.stack([eft, par, nxt], axis=1).astype(jnp.int32)

    x_sorted = _sc_dispatch(x, pos.astype(jnp.int32))
    out_sorted = _ffn(sched, x_sorted, gate_w, gate_b,
                      up_w, up_b, down_w, down_b)
    pos_cat = jnp.concatenate([pos[0::TOP_K], pos[1::TOP_K]]).astype(jnp.int32)
    g = _sc_row_gather(out_sorted, pos_cat, TOP_K * T)
    nxt = _combine_add(g, top_p)
    return nxt.reshape(hidden_states.shape), score


# speedup vs baseline: 1.0088x; 1.0088x over previous
import functools

import jax
import jax.numpy as jnp
from jax import lax
from jax.experimental import pallas as pl
from jax.experimental.pallas import tpu as pltpu
from jax.experimental.pallas import tpu_sc as plsc

E = 8
TOP_K = 2
H = 1024
FF = 2048
ALPHA = 1.702
LIMIT = 7.0
T = 2048

BM = 128
P_PAD = TOP_K * T + E * BM
M_TILES = P_PAD // BM

NC, NS = 2, 16
NW = NC * NS
GCHUNK = 32
H_HALF = H // 2


def _router_body(x_ref, rw_ref, rb_ref, score_ref, idx_ref, p_ref):
    x = x_ref[...]
    logits = lax.dot_general(x, rw_ref[...], (((1,), (1,)), ((), ())),
                             preferred_element_type=jnp.float32) + rb_ref[...]
    iota = lax.broadcasted_iota(jnp.int32, (T, E), 1)
    m1 = jnp.max(logits, axis=1, keepdims=True)
    a1 = jnp.min(jnp.where(logits == m1, iota, E), axis=1, keepdims=True)
    l2 = jnp.where(iota == a1, -jnp.inf, logits)
    m2 = jnp.max(l2, axis=1, keepdims=True)
    a2 = jnp.min(jnp.where(l2 == m2, iota, E), axis=1, keepdims=True)
    ex = jnp.exp(m2 - m1)
    p1 = 1.0 / (1.0 + ex)
    p2 = ex / (1.0 + ex)
    score_ref[...] = jnp.where(iota == a1, p1, 0.0) + jnp.where(iota == a2, p2, 0.0)
    idx_ref[...] = jnp.concatenate([a1, a2], axis=1)
    p_ref[...] = jnp.concatenate([p1, p2], axis=1)


def _router(x, router_w, router_b):
    return pl.pallas_call(
        _router_body,
        out_shape=(
            jax.ShapeDtypeStruct((T, E), jnp.float32),
            jax.ShapeDtypeStruct((T, TOP_K), jnp.int32),
            jax.ShapeDtypeStruct((T, TOP_K), jnp.float32),
        ),
    )(x, router_w, router_b)


def _sc_row_gather(table, idx, n_rows):
    mesh = plsc.VectorSubcoreMesh(core_axis_name="c", subcore_axis_name="s",
                                  num_cores=NC, num_subcores=NS)
    rows_per_w = n_rows // NW

    nchunks = rows_per_w // GCHUNK

    @functools.partial(
        pl.kernel,
        out_type=jax.ShapeDtypeStruct((n_rows, H), jnp.float32),
        mesh=mesh,
        scratch_types=[
            [pltpu.VMEM((GCHUNK,), jnp.int32)] * 2,
            [pltpu.VMEM((GCHUNK, H), jnp.float32)] * 2,
            [pltpu.SemaphoreType.DMA] * 2,
            [pltpu.SemaphoreType.DMA] * 2,
        ],
    )
    def k(x_hbm, i_hbm, out_hbm, idx_v, rows_v, gsem, wsem):
        wid = lax.axis_index("s") * NC + lax.axis_index("c")
        base0 = wid * rows_per_w

        gd = [None, None]
        wd = [None, None]
        for i in range(nchunks):
            b = i % 2
            base = base0 + i * GCHUNK
            if i >= 2:
                wd[b].wait()
            pltpu.sync_copy(i_hbm.at[pl.ds(base, GCHUNK)], idx_v[b])
            gd[b] = pltpu.async_copy(x_hbm.at[idx_v[b]], rows_v[b], gsem[b])
            if i >= 1:
                pb = (i - 1) % 2
                gd[pb].wait()
                wd[pb] = pltpu.async_copy(
                    rows_v[pb],
                    out_hbm.at[pl.ds(base0 + (i - 1) * GCHUNK, GCHUNK)],
                    wsem[pb])
        lb = (nchunks - 1) % 2
        gd[lb].wait()
        pltpu.sync_copy(rows_v[lb],
                        out_hbm.at[pl.ds(base0 + (nchunks - 1) * GCHUNK, GCHUNK)])
        if nchunks >= 2:
            wd[(nchunks - 2) % 2].wait()

    return k(table, idx)


def _sc_dispatch(x, pos):
    mesh = plsc.VectorSubcoreMesh(core_axis_name="c", subcore_axis_name="s",
                                  num_cores=NC, num_subcores=NS)
    n_pairs = TOP_K * T
    rows_per_w = n_pairs // NW
    nchunks = rows_per_w // GCHUNK

    @functools.partial(
        pl.kernel,
        out_type=jax.ShapeDtypeStruct((P_PAD, H), jnp.float32),
        mesh=mesh,
        scratch_types=[
            [pltpu.VMEM((GCHUNK,), jnp.int32)] * 2,
            [pltpu.VMEM((GCHUNK,), jnp.int32)] * 2,
            [pltpu.VMEM((GCHUNK, H), jnp.float32)] * 2,
            [pltpu.SemaphoreType.DMA] * 2,
            [pltpu.SemaphoreType.DMA] * 2,
        ],
    )
    def k(x_hbm, pos_hbm, out_hbm, pos_v, idx_v, rows_v, gsem, wsem):
        wid = lax.axis_index("s") * NC + lax.axis_index("c")
        base0 = wid * rows_per_w
        gd = [None, None]
        wd = [None, None]
        for i in range(nchunks):
            b = i % 2
            base = base0 + i * GCHUNK
            if i >= 2:
                wd[b].wait()
            pltpu.sync_copy(pos_hbm.at[pl.ds(base, GCHUNK)], pos_v[b])
            v = lax.iota(jnp.int32, 16)
            for h in range(GCHUNK // 16):
                idx_v[b][pl.ds(h * 16, 16)] = lax.shift_right_logical(
                    v + (base + h * 16), 1)
            gd[b] = pltpu.async_copy(x_hbm.at[idx_v[b]], rows_v[b], gsem[b])
            if i >= 1:
                pb = (i - 1) % 2
                gd[pb].wait()
                wd[pb] = pltpu.async_copy(rows_v[pb], out_hbm.at[pos_v[pb]],
                                          wsem[pb])
        lb = (nchunks - 1) % 2
        gd[lb].wait()
        pltpu.sync_copy(rows_v[lb], out_hbm.at[pos_v[lb]])
        if nchunks >= 2:
            wd[(nchunks - 2) % 2].wait()

    return k(x, pos)


def _ffn_body(s_ref, x_ref, gw_hbm, gb_ref, uw_hbm, ub_ref,
              dw_hbm, db_ref, o_ref, gw_v, uw_v, dw_v, sems):
    i = pl.program_id(0)
    e = s_ref[i, 0]
    p = s_ref[i, 1]
    nx = s_ref[i, 2]
    prev = s_ref[jnp.maximum(i - 1, 0), 0]
    changed = jnp.logical_or(i == 0, e != prev)

    def start(ee, bb):
        pltpu.make_async_copy(gw_hbm.at[ee], gw_v.at[bb], sems.at[bb, 0]).start()
        pltpu.make_async_copy(uw_hbm.at[ee], uw_v.at[bb], sems.at[bb, 1]).start()
        pltpu.make_async_copy(dw_hbm.at[ee], dw_v.at[bb], sems.at[bb, 2]).start()

    def wait(ee, bb):
        pltpu.make_async_copy(gw_hbm.at[ee], gw_v.at[bb], sems.at[bb, 0]).wait()
        pltpu.make_async_copy(uw_hbm.at[ee], uw_v.at[bb], sems.at[bb, 1]).wait()
        pltpu.make_async_copy(dw_hbm.at[ee], dw_v.at[bb], sems.at[bb, 2]).wait()

    @pl.when(i == 0)
    def _():
        start(e, p)

    @pl.when(changed)
    def _():
        wait(e, p)

        @pl.when(nx != e)
        def _():
            start(nx, 1 - p)

    x = x_ref[...]
    gate = lax.dot_general(x, gw_v[p], (((1,), (1,)), ((), ())),
                           preferred_element_type=jnp.float32) + gb_ref[0]
    gate = jnp.minimum(gate, LIMIT)
    glu = gate * jax.nn.sigmoid(gate * ALPHA)
    up = lax.dot_general(x, uw_v[p], (((1,), (1,)), ((), ())),
                         preferred_element_type=jnp.float32) + ub_ref[0]
    up = jnp.clip(up, -LIMIT, LIMIT)
    gated = (up + 1.0) * glu
    out = lax.dot_general(gated, dw_v[p], (((1,), (1,)), ((), ())),
                          preferred_element_type=jnp.float32) + db_ref[0]
    o_ref[...] = out


def _ffn(sched, x_sorted, gate_w, gate_b, up_w, up_b, down_w, down_b):
    grid_spec = pltpu.PrefetchScalarGridSpec(
        num_scalar_prefetch=1,
        grid=(M_TILES,),
        in_specs=[
            pl.BlockSpec((BM, H), lambda i, s: (i, 0)),
            pl.BlockSpec(memory_space=pl.ANY),
            pl.BlockSpec((1, 1, FF), lambda i, s: (s[i, 0], 0, 0)),
            pl.BlockSpec(memory_space=pl.ANY),
            pl.BlockSpec((1, 1, FF), lambda i, s: (s[i, 0], 0, 0)),
            pl.BlockSpec(memory_space=pl.ANY),
            pl.BlockSpec((1, 1, H), lambda i, s: (s[i, 0], 0, 0)),
        ],
        out_specs=pl.BlockSpec((BM, H), lambda i, s: (i, 0)),
        scratch_shapes=[
            pltpu.VMEM((2, FF, H), jnp.float32),
            pltpu.VMEM((2, FF, H), jnp.float32),
            pltpu.VMEM((2, H, FF), jnp.float32),
            pltpu.SemaphoreType.DMA((2, 3)),
        ],
    )
    return pl.pallas_call(
        _ffn_body,
        grid_spec=grid_spec,
        out_shape=jax.ShapeDtypeStruct((P_PAD, H), jnp.float32),
    )(sched, x_sorted,
      gate_w, gate_b.reshape(E, 1, FF),
      up_w, up_b.reshape(E, 1, FF),
      down_w, down_b.reshape(E, 1, H))


BN_ADD = 256


def _add_body(a_ref, b_ref, p_ref, o_ref):
    p = p_ref[...]
    o_ref[...] = a_ref[...] * p[:, 0:1] + b_ref[...] * p[:, 1:2]


def _combine_add(g, top_p):
    nblk = T // BN_ADD
    return pl.pallas_call(
        _add_body,
        grid=(nblk,),
        in_specs=[
            pl.BlockSpec((BN_ADD, H), lambda i: (i, 0)),
            pl.BlockSpec((BN_ADD, H), lambda i: (i + nblk, 0)),
            pl.BlockSpec((BN_ADD, TOP_K), lambda i: (i, 0)),
        ],
        out_specs=pl.BlockSpec((BN_ADD, H), lambda i: (i, 0)),
        out_shape=jax.ShapeDtypeStruct((T, H), jnp.float32),
    )(g, g, top_p)


def kernel(hidden_states, router_w, router_b, gate_w, gate_b, up_w, up_b,
           down_w, down_b):
    x = hidden_states.reshape(T, H)
    score, top_idx, top_p = _router(x, router_w, router_b.reshape(1, E))

    pairs = top_idx.reshape(-1)
    onehot = (pairs[:, None] == jnp.arange(E)[None, :]).astype(jnp.int32)
    counts = onehot.sum(0)
    rank = jnp.take_along_axis(jnp.cumsum(onehot, axis=0) - onehot,
                               pairs[:, None], axis=1)[:, 0]
    aligned = ((counts + BM - 1) // BM) * BM
    bounds = jnp.cumsum(aligned)
    off = bounds - aligned
    pos = off[pairs] + rank
    tile_start = jnp.arange(M_TILES, dtype=jnp.int32) * BM
    eft = jnp.minimum(
        (tile_start[:, None] >= bounds[None, :]).astype(jnp.int32).sum(1), E - 1)
    changes = jnp.concatenate([jnp.ones((1,), bool), eft[1:] != eft[:-1]])
    seg_id = jnp.cumsum(changes.astype(jnp.int32)) - 1
    par = seg_id % 2
    seg_expert = jnp.zeros((M_TILES,), jnp.int32).at[seg_id].set(eft)
    nxt = seg_expert[jnp.minimum(seg_id + 1, seg_id[-1])]
    sched = jnp.stack([eft, par, nxt], axis=1).astype(jnp.int32)

    x_sorted = _sc_dispatch(x, pos.astype(jnp.int32))
    out_sorted = _ffn(sched, x_sorted, gate_w, gate_b,
                      up_w, up_b, down_w, down_b)
    pos_cat = jnp.concatenate([pos[0::TOP_K], pos[1::TOP_K]]).astype(jnp.int32)
    g = _sc_row_gather(out_sorted, pos_cat, TOP_K * T)
    nxt = _combine_add(g, top_p)
    return nxt.reshape(hidden_states.shape), score
